# baseline (speedup 1.0000x reference)
import math

import jax
import jax.numpy as jnp
import numpy as np
from jax import lax
from jax.experimental import pallas as pl
from jax.experimental.pallas import tpu as pltpu
from jax.experimental.pallas import tpu_sc as plsc

_NC = 2
_NS = 16
_NW = _NC * _NS
_LANES = 16


def _make_pe(seq_len: int, emb: int) -> np.ndarray:
    pe = np.zeros((seq_len, emb), dtype=np.float32)
    position = np.arange(0, seq_len, dtype=np.float32)[:, None]
    div_term = np.exp(
        np.arange(0, emb, 2, dtype=np.float32) * -(math.log(10000.0) / emb)
    )
    pe[:, 0::2] = np.sin(position * div_term)
    pe[:, 1::2] = np.cos(position * div_term)
    return pe


def _build(seq: int, batch: int, vocab: int, emb: int):
    assert batch % _NW == 0 and seq % 2 == 0 and emb == 64 and vocab % 2 == 0
    bw = batch // _NW
    vregs = emb // _LANES
    scale = float(math.sqrt(emb))
    nblk = bw // _LANES
    mesh = plsc.VectorSubcoreMesh(core_axis_name="c", subcore_axis_name="s")

    @jax.jit
    def run(src, table128, pe_flat):
        def body(src_hbm, pe_hbm, table_hbm, out_hbm,
                 pe_v, ib0, ib1, qb0, qb1, r0, r1, o0, o1,
                 isem0, isem1, g0, g1, st0, st1):
            wid = lax.axis_index("s") * _NC + lax.axis_index("c")
            boff = wid * bw
            pltpu.sync_copy(pe_hbm, pe_v)

            ibuf = (ib0, ib1)
            qbuf = (qb0, qb1)
            rows = (r0, r1)
            outv = (o0, o1)
            isem = (isem0, isem1)
            gsem = (g0, g1)
            ssem = (st0, st1)

            def idx_start(s, b):
                pltpu.async_copy(
                    src_hbm.at[s, pl.ds(boff, bw)], ibuf[b], isem[b]
                )

            def idx_wait(b):
                pltpu.make_async_copy(
                    src_hbm.at[0, pl.ds(boff, bw)], ibuf[b], isem[b]
                ).wait()

            def mangle_and_gather(b):
                for jb in range(nblk):
                    sl = pl.ds(jb * _LANES, _LANES)
                    v = ibuf[b][sl]
                    qbuf[b][sl] = lax.shift_right_logical(v, 1)
                pltpu.async_copy(table_hbm.at[qbuf[b]], rows[b], gsem[b])

            def gather_wait(b):
                pltpu.make_async_copy(
                    table_hbm.at[qbuf[b]], rows[b], gsem[b]
                ).wait()

            def store_start(s, b):
                pltpu.async_copy(
                    outv[b], out_hbm.at[s, :, pl.ds(boff, bw)], ssem[b]
                )

            def store_wait(b):
                pltpu.make_async_copy(
                    outv[b], out_hbm.at[0, :, pl.ds(boff, bw)], ssem[b]
                ).wait()

            lane_iota = lax.iota(jnp.int32, _LANES)
            row_iotas = [lane_iota + jb * _LANES for jb in range(nblk)]

            def compute(s, b):
                rbuf = rows[b]
                obuf = outv[b]
                hvecs = [
                    lax.shift_left(
                        lax.bitwise_and(
                            ibuf[b][pl.ds(jb * _LANES, _LANES)], 1
                        ),
                        6,
                    )
                    for jb in range(nblk)
                ]

                @pl.loop(0, emb, unroll=2)
                def _col(c):
                    pe_c = plsc.load_gather(
                        pe_v, [jnp.full((_LANES,), s * emb + c, jnp.int32)]
                    )
                    for jb in range(nblk):
                        vals = plsc.load_gather(
                            rbuf, [row_iotas[jb], hvecs[jb] + c]
                        )
                        obuf[c, pl.ds(jb * _LANES, _LANES)] = (
                            vals * scale + pe_c
                        )

            idx_start(0, 0)
            idx_wait(0)
            mangle_and_gather(0)

            @pl.loop(0, seq, step=2)
            def _iter(g):
                for b in range(2):
                    s = g + b
                    nxt = 1 - b

                    @pl.when(s + 1 < seq)
                    def _prefetch():
                        idx_start(s + 1, nxt)

                        @pl.when(s >= 1)
                        def _drain():
                            store_wait(nxt)
                        idx_wait(nxt)
                        mangle_and_gather(nxt)

                    gather_wait(b)
                    compute(s, b)
                    store_start(s, b)

            store_wait(0)
            store_wait(1)

        return pl.kernel(
            body,
            out_type=jax.ShapeDtypeStruct((seq, emb, batch), jnp.float32),
            mesh=mesh,
            scratch_types=[
                pltpu.VMEM((seq * emb,), jnp.float32),
                pltpu.VMEM((bw,), jnp.int32),
                pltpu.VMEM((bw,), jnp.int32),
                pltpu.VMEM((bw,), jnp.int32),
                pltpu.VMEM((bw,), jnp.int32),
                pltpu.VMEM((bw, 2 * emb), jnp.float32),
                pltpu.VMEM((bw, 2 * emb), jnp.float32),
                pltpu.VMEM((emb, bw), jnp.float32),
                pltpu.VMEM((emb, bw), jnp.float32),
                pltpu.SemaphoreType.DMA,
                pltpu.SemaphoreType.DMA,
                pltpu.SemaphoreType.DMA,
                pltpu.SemaphoreType.DMA,
                pltpu.SemaphoreType.DMA,
                pltpu.SemaphoreType.DMA,
            ],
            compiler_params=pltpu.CompilerParams(needs_layout_passes=False),
        )(src, pe_flat, table128)

    return run


def kernel(src, table, step=0):
    seq, batch = src.shape
    vocab, emb = table.shape
    run = _build(seq, batch, vocab, emb)
    pe_flat = jnp.asarray(_make_pe(seq, emb).reshape(-1))
    table128 = table.reshape(vocab // 2, 2 * emb)
    out_t = run(src.astype(jnp.int32), table128, pe_flat)
    return jnp.swapaxes(out_t, 1, 2)


# speedup vs baseline: 1.0014x; 1.0014x over previous
import math

import jax
import jax.numpy as jnp
import numpy as np
from jax import lax
from jax.experimental import pallas as pl
from jax.experimental.pallas import tpu as pltpu
from jax.experimental.pallas import tpu_sc as plsc

_NC = 2
_NS = 16
_NW = _NC * _NS
_LANES = 16


def _make_pe(seq_len: int, emb: int) -> np.ndarray:
    pe = np.zeros((seq_len, emb), dtype=np.float32)
    position = np.arange(0, seq_len, dtype=np.float32)[:, None]
    div_term = np.exp(
        np.arange(0, emb, 2, dtype=np.float32) * -(math.log(10000.0) / emb)
    )
    pe[:, 0::2] = np.sin(position * div_term)
    pe[:, 1::2] = np.cos(position * div_term)
    return pe


def _build(seq: int, batch: int, vocab: int, emb: int):
    assert batch % _NW == 0 and seq % 2 == 0 and emb == 64 and vocab % 2 == 0
    bw = batch // _NW
    vregs = emb // _LANES
    scale = float(math.sqrt(emb))
    nblk = bw // _LANES
    mesh = plsc.VectorSubcoreMesh(core_axis_name="c", subcore_axis_name="s")

    @jax.jit
    def run(src, table128, pe_flat):
        def body(src_hbm, pe_hbm, table_hbm, out_hbm,
                 pe_v, ib0, ib1, qb0, qb1, r0, r1, o0, o1,
                 isem0, isem1, g0, g1, st0, st1):
            wid = lax.axis_index("s") * _NC + lax.axis_index("c")
            boff = wid * bw
            pltpu.sync_copy(pe_hbm, pe_v)

            ibuf = (ib0, ib1)
            qbuf = (qb0, qb1)
            rows = (r0, r1)
            outv = (o0, o1)
            isem = (isem0, isem1)
            gsem = (g0, g1)
            ssem = (st0, st1)

            def idx_start(s, b):
                pltpu.async_copy(
                    src_hbm.at[s, pl.ds(boff, bw)], ibuf[b], isem[b]
                )

            def idx_wait(b):
                pltpu.make_async_copy(
                    src_hbm.at[0, pl.ds(boff, bw)], ibuf[b], isem[b]
                ).wait()

            def gather_start(b):
                for jb in range(nblk):
                    sl = pl.ds(jb * _LANES, _LANES)
                    qbuf[b][sl] = lax.shift_right_logical(ibuf[b][sl], 1)
                pltpu.async_copy(table_hbm.at[qbuf[b]], rows[b], gsem[b])

            def gather_wait(b):
                pltpu.make_async_copy(
                    table_hbm.at[qbuf[b]], rows[b], gsem[b]
                ).wait()

            def store_start(s, b):
                pltpu.async_copy(
                    outv[b], out_hbm.at[s, :, pl.ds(boff, bw)], ssem[b]
                )

            def store_wait(b):
                pltpu.make_async_copy(
                    outv[b], out_hbm.at[0, :, pl.ds(boff, bw)], ssem[b]
                ).wait()

            lane_iota = lax.iota(jnp.int32, _LANES)
            row_iotas = [lane_iota + jb * _LANES for jb in range(nblk)]

            def compute(s, b):
                rbuf = rows[b]
                obuf = outv[b]
                hvecs = [
                    lax.shift_left(
                        lax.bitwise_and(
                            ibuf[b][pl.ds(jb * _LANES, _LANES)], 1
                        ),
                        6,
                    )
                    for jb in range(nblk)
                ]

                @pl.loop(0, emb, unroll=2)
                def _col(c):
                    pe_c = plsc.load_gather(
                        pe_v, [jnp.full((_LANES,), s * emb + c, jnp.int32)]
                    )
                    for jb in range(nblk):
                        vals = plsc.load_gather(
                            rbuf, [row_iotas[jb], hvecs[jb] + c]
                        )
                        obuf[c, pl.ds(jb * _LANES, _LANES)] = (
                            vals * scale + pe_c
                        )

            idx_start(0, 0)
            idx_wait(0)
            gather_start(0)

            @pl.loop(0, seq, step=2)
            def _iter(g):
                for b in range(2):
                    s = g + b
                    nxt = 1 - b

                    @pl.when(s + 1 < seq)
                    def _prefetch():
                        idx_start(s + 1, nxt)

                        @pl.when(s >= 1)
                        def _drain():
                            store_wait(nxt)
                        idx_wait(nxt)
                        gather_start(nxt)

                    gather_wait(b)
                    compute(s, b)
                    store_start(s, b)

            store_wait(0)
            store_wait(1)

        return pl.kernel(
            body,
            out_type=jax.ShapeDtypeStruct((seq, emb, batch), jnp.float32),
            mesh=mesh,
            scratch_types=[
                pltpu.VMEM((seq * emb,), jnp.float32),
                pltpu.VMEM((bw,), jnp.int32),
                pltpu.VMEM((bw,), jnp.int32),
                pltpu.VMEM((bw,), jnp.int32),
                pltpu.VMEM((bw,), jnp.int32),
                pltpu.VMEM((bw, 2 * emb), jnp.float32),
                pltpu.VMEM((bw, 2 * emb), jnp.float32),
                pltpu.VMEM((emb, bw), jnp.float32),
                pltpu.VMEM((emb, bw), jnp.float32),
                pltpu.SemaphoreType.DMA,
                pltpu.SemaphoreType.DMA,
                pltpu.SemaphoreType.DMA,
                pltpu.SemaphoreType.DMA,
                pltpu.SemaphoreType.DMA,
                pltpu.SemaphoreType.DMA,
            ],
            compiler_params=pltpu.CompilerParams(needs_layout_passes=False),
        )(src, pe_flat, table128)

    return run


def kernel(src, table, step=0):
    seq, batch = src.shape
    vocab, emb = table.shape
    run = _build(seq, batch, vocab, emb)
    pe_flat = jnp.asarray(_make_pe(seq, emb).reshape(-1))
    table128 = table.reshape(vocab // 2, 2 * emb)
    out_t = run(src.astype(jnp.int32), table128, pe_flat)
    return jnp.swapaxes(out_t, 1, 2)


# speedup vs baseline: 1.3308x; 1.3288x over previous
import math

import jax
import jax.numpy as jnp
import numpy as np
from jax import lax
from jax.experimental import pallas as pl
from jax.experimental.pallas import tpu as pltpu
from jax.experimental.pallas import tpu_sc as plsc

_NC = 2
_NS = 16
_NW = _NC * _NS
_LANES = 16


def _make_pe(seq_len: int, emb: int) -> np.ndarray:
    pe = np.zeros((seq_len, emb), dtype=np.float32)
    position = np.arange(0, seq_len, dtype=np.float32)[:, None]
    div_term = np.exp(
        np.arange(0, emb, 2, dtype=np.float32) * -(math.log(10000.0) / emb)
    )
    pe[:, 0::2] = np.sin(position * div_term)
    pe[:, 1::2] = np.cos(position * div_term)
    return pe


def _build(seq: int, batch: int, vocab: int, emb: int):
    assert batch % _NW == 0 and seq % 2 == 0 and emb == 64 and vocab % 2 == 0
    bw = batch // _NW
    vregs = emb // _LANES
    scale = float(math.sqrt(emb))
    nblk = bw // _LANES
    mesh = plsc.VectorSubcoreMesh(core_axis_name="c", subcore_axis_name="s")

    @jax.jit
    def run(src, table128, pe_flat):
        def body(src_hbm, pe_hbm, table_hbm, out_hbm,
                 pe_v, ib0, ib1, qb0, qb1, r0, r1, o0, o1,
                 isem0, isem1, g0, g1, st0, st1):
            wid = lax.axis_index("s") * _NC + lax.axis_index("c")
            boff = wid * bw
            pltpu.sync_copy(pe_hbm, pe_v)

            ibuf = (ib0, ib1)
            qbuf = (qb0, qb1)
            rows = (r0, r1)
            outv = (o0, o1)
            isem = (isem0, isem1)
            gsem = (g0, g1)
            ssem = (st0, st1)

            def idx_start(s, b):
                pltpu.async_copy(
                    src_hbm.at[s, pl.ds(boff, bw)], ibuf[b], isem[b]
                )

            def idx_wait(b):
                pltpu.make_async_copy(
                    src_hbm.at[0, pl.ds(boff, bw)], ibuf[b], isem[b]
                ).wait()

            def gather_start(b):
                for jb in range(nblk):
                    sl = pl.ds(jb * _LANES, _LANES)
                    qbuf[b][sl] = lax.shift_right_logical(ibuf[b][sl], 1)
                pltpu.async_copy(table_hbm.at[qbuf[b]], rows[b], gsem[b])

            def gather_wait(b):
                pltpu.make_async_copy(
                    table_hbm.at[qbuf[b]], rows[b], gsem[b]
                ).wait()

            def store_start(s, b):
                pltpu.async_copy(
                    outv[b], out_hbm.at[s, pl.ds(boff, bw), :], ssem[b]
                )

            def store_wait(b):
                pltpu.make_async_copy(
                    outv[b], out_hbm.at[0, pl.ds(boff, bw), :], ssem[b]
                ).wait()

            def compute(s, b):
                rbuf = rows[b]
                obuf = outv[b]
                pech = [
                    pe_v[pl.ds(s * emb + k * _LANES, _LANES)]
                    for k in range(vregs)
                ]

                @pl.loop(0, nblk)
                def _blk(jb):
                    iv = ibuf[b][pl.ds(jb * _LANES, _LANES)]
                    pv = lax.shift_left(lax.bitwise_and(iv, 1), 6)
                    for l in range(_LANES):
                        j = jb * _LANES + l
                        p = pv[l]
                        for k in range(vregs):
                            vals = rbuf[j, pl.ds(p + k * _LANES, _LANES)]
                            obuf[j, pl.ds(k * _LANES, _LANES)] = (
                                vals * scale + pech[k]
                            )

            idx_start(0, 0)
            idx_wait(0)
            gather_start(0)

            @pl.loop(0, seq, step=2)
            def _iter(g):
                for b in range(2):
                    s = g + b
                    nxt = 1 - b

                    @pl.when(s + 1 < seq)
                    def _prefetch():
                        idx_start(s + 1, nxt)

                        @pl.when(s >= 1)
                        def _drain():
                            store_wait(nxt)
                        idx_wait(nxt)
                        gather_start(nxt)

                    gather_wait(b)
                    compute(s, b)
                    store_start(s, b)

            store_wait(0)
            store_wait(1)

        return pl.kernel(
            body,
            out_type=jax.ShapeDtypeStruct((seq, batch, emb), jnp.float32),
            mesh=mesh,
            scratch_types=[
                pltpu.VMEM((seq * emb,), jnp.float32),
                pltpu.VMEM((bw,), jnp.int32),
                pltpu.VMEM((bw,), jnp.int32),
                pltpu.VMEM((bw,), jnp.int32),
                pltpu.VMEM((bw,), jnp.int32),
                pltpu.VMEM((bw, 2 * emb), jnp.float32),
                pltpu.VMEM((bw, 2 * emb), jnp.float32),
                pltpu.VMEM((bw, emb), jnp.float32),
                pltpu.VMEM((bw, emb), jnp.float32),
                pltpu.SemaphoreType.DMA,
                pltpu.SemaphoreType.DMA,
                pltpu.SemaphoreType.DMA,
                pltpu.SemaphoreType.DMA,
                pltpu.SemaphoreType.DMA,
                pltpu.SemaphoreType.DMA,
            ],
            compiler_params=pltpu.CompilerParams(needs_layout_passes=False),
        )(src, pe_flat, table128)

    return run


def kernel(src, table, step=0):
    seq, batch = src.shape
    vocab, emb = table.shape
    run = _build(seq, batch, vocab, emb)
    pe_flat = jnp.asarray(_make_pe(seq, emb).reshape(-1))
    table128 = table.reshape(vocab // 2, 2 * emb)
    return run(src.astype(jnp.int32), table128, pe_flat)
